# baseline (speedup 1.0000x reference)
import functools

import jax
import jax.numpy as jnp
from jax import lax
from jax.experimental import pallas as pl
from jax.experimental.pallas import tpu as pltpu
from jax.experimental.pallas import tpu_sc as plsc

B = 4096
L = 20
EMB = 128
GEMB = 32
GV = 1000
H1 = 256
H2 = 128

NC = 2
NS = 16
NW = NC * NS
BW = B // NW


def _sc_gather_pool(mi, ui, mg, mtab, utab, gtab):
  mesh = plsc.VectorSubcoreMesh(core_axis_name="c", subcore_axis_name="s")

  @functools.partial(
      pl.kernel,
      mesh=mesh,
      compiler_params=pltpu.CompilerParams(
          use_tc_tiling_on_sc=False, needs_layout_passes=False),
      out_type=[
          jax.ShapeDtypeStruct((B, EMB), jnp.float32),
          jax.ShapeDtypeStruct((B, EMB), jnp.float32),
          jax.ShapeDtypeStruct((B, EMB), jnp.float32),
      ],
      scratch_types=[
          pltpu.VMEM((BW,), jnp.int32),
          pltpu.VMEM((BW,), jnp.int32),
          pltpu.VMEM((BW, L), jnp.int32),
          pltpu.VMEM((GV, GEMB), jnp.float32),
          pltpu.VMEM((BW, EMB), jnp.float32),
          pltpu.VMEM((BW, EMB), jnp.float32),
          pltpu.VMEM((BW, GEMB), jnp.float32),
          pltpu.SemaphoreType.DMA,
          pltpu.SemaphoreType.DMA,
          pltpu.SemaphoreType.DMA,
          pltpu.SemaphoreType.DMA,
      ],
  )
  def k(mi_hbm, ui_hbm, mg_hbm, mtab_hbm, utab_hbm, gtab_hbm,
        mout_hbm, uout_hbm, gout_hbm,
        mi_v, ui_v, gi_v, tab_v, mrows, urows, gavg,
        sem_m, sem_u, sem_t, sem_i):
    wid = lax.axis_index("s") * NC + lax.axis_index("c")
    base = wid * BW

    cp_t = pltpu.async_copy(gtab_hbm, tab_v, sem_t)
    cp_i1 = pltpu.async_copy(mi_hbm.at[pl.ds(base, BW)], mi_v, sem_i)
    cp_i2 = pltpu.async_copy(ui_hbm.at[pl.ds(base, BW)], ui_v, sem_i)
    cp_i3 = pltpu.async_copy(mg_hbm.at[pl.ds(base, BW)], gi_v, sem_i)

    cp_i1.wait()
    cp_m = pltpu.async_copy(mtab_hbm.at[mi_v], mrows, sem_m)
    cp_i2.wait()
    cp_u = pltpu.async_copy(utab_hbm.at[ui_v], urows, sem_u)
    cp_i3.wait()
    cp_t.wait()

    one = jnp.ones((16,), jnp.float32)
    fzero = jnp.zeros((16,), jnp.float32)
    lanes = lax.iota(jnp.int32, 16)
    hi = lanes + 16

    def one_row(i):
      iv = jnp.full((16,), i, jnp.int32)
      acc0 = jnp.zeros((16,), jnp.float32)
      acc1 = jnp.zeros((16,), jnp.float32)
      cnt = jnp.zeros((16,), jnp.float32)
      for l in range(L):
        ids = plsc.load_gather(gi_v, [iv, jnp.full((16,), l, jnp.int32)])
        acc0 = acc0 + plsc.load_gather(tab_v, [ids, lanes])
        acc1 = acc1 + plsc.load_gather(tab_v, [ids, hi])
        cnt = cnt + jnp.where(ids != 0, one, fzero)
      r = 1.0 / cnt
      gavg[i, pl.ds(0, 16)] = acc0 * r
      gavg[i, pl.ds(16, 16)] = acc1 * r

    def pool_body(i2, c):
      one_row(i2 * 2)
      one_row(i2 * 2 + 1)
      return c

    lax.fori_loop(0, BW // 2, pool_body, 0)

    cp_m.wait()
    pltpu.sync_copy(mrows, mout_hbm.at[pl.ds(base, BW)])
    cp_u.wait()
    pltpu.sync_copy(urows, uout_hbm.at[pl.ds(base, BW)])
    pltpu.sync_copy(gavg, gout_hbm.at[pl.ds(base, BW), pl.ds(0, GEMB)])

  return k(mi, ui, mg, mtab, utab, gtab)


def _mlp_body(m_ref, u_ref, g_ref, w1_ref, b1_ref, w2_ref, b2_ref, out_ref):
  h = jnp.dot(m_ref[...], w1_ref[0:EMB, :], preferred_element_type=jnp.float32)
  h = h + jnp.dot(u_ref[...], w1_ref[EMB:2 * EMB, :],
                  preferred_element_type=jnp.float32)
  h = h + jnp.dot(g_ref[:, 0:GEMB], w1_ref[2 * EMB:, :],
                  preferred_element_type=jnp.float32)
  h = jnp.maximum(h + b1_ref[...], 0.0)
  o = jnp.dot(h, w2_ref[...], preferred_element_type=jnp.float32)
  out_ref[...] = jnp.maximum(o + b2_ref[...], 0.0)


def _mlp(movie_e, user_e, genre_avg, W1, b1, W2, b2):
  BB = 2048
  grid = (B // BB,)
  return pl.pallas_call(
      _mlp_body,
      grid=grid,
      in_specs=[
          pl.BlockSpec((BB, EMB), lambda i: (i, 0)),
          pl.BlockSpec((BB, EMB), lambda i: (i, 0)),
          pl.BlockSpec((BB, EMB), lambda i: (i, 0)),
          pl.BlockSpec((EMB + EMB + GEMB, H1), lambda i: (0, 0)),
          pl.BlockSpec((H1,), lambda i: (0,)),
          pl.BlockSpec((H1, H2), lambda i: (0, 0)),
          pl.BlockSpec((H2,), lambda i: (0,)),
      ],
      out_specs=pl.BlockSpec((BB, H2), lambda i: (i, 0)),
      out_shape=jax.ShapeDtypeStruct((B, H2), jnp.float32),
  )(movie_e, user_e, genre_avg, W1, b1, W2, b2)


def kernel(movie_id, user_id, movie_genres, movie_table, user_table,
           genre_table, W1, b1, W2, b2):
  movie_e, user_e, genre_avg = _sc_gather_pool(
      movie_id, user_id, movie_genres, movie_table, user_table, genre_table)

  return _mlp(movie_e, user_e, genre_avg, W1, b1, W2, b2)


# speedup vs baseline: 1.0046x; 1.0046x over previous
import functools

import jax
import jax.numpy as jnp
from jax import lax
from jax.experimental import pallas as pl
from jax.experimental.pallas import tpu as pltpu
from jax.experimental.pallas import tpu_sc as plsc

B = 4096
L = 20
EMB = 128
GEMB = 32
GV = 1000
H1 = 256
H2 = 128

NC = 2
NS = 16
NW = NC * NS
BW = B // NW


def _sc_gather_pool(mi, ui, mg, mtab, utab, gtab):
  mesh = plsc.VectorSubcoreMesh(core_axis_name="c", subcore_axis_name="s")

  @functools.partial(
      pl.kernel,
      mesh=mesh,
      compiler_params=pltpu.CompilerParams(
          use_tc_tiling_on_sc=False, needs_layout_passes=False),
      out_type=[
          jax.ShapeDtypeStruct((B, EMB), jnp.float32),
          jax.ShapeDtypeStruct((B, EMB), jnp.float32),
          jax.ShapeDtypeStruct((B, EMB), jnp.float32),
      ],
      scratch_types=[
          pltpu.VMEM((BW,), jnp.int32),
          pltpu.VMEM((BW,), jnp.int32),
          pltpu.VMEM((BW, L), jnp.int32),
          pltpu.VMEM((GV, GEMB), jnp.float32),
          pltpu.VMEM((BW, EMB), jnp.float32),
          pltpu.VMEM((BW, EMB), jnp.float32),
          pltpu.VMEM((BW, GEMB), jnp.float32),
          pltpu.SemaphoreType.DMA,
          pltpu.SemaphoreType.DMA,
          pltpu.SemaphoreType.DMA,
          pltpu.SemaphoreType.DMA,
      ],
  )
  def k(mi_hbm, ui_hbm, mg_hbm, mtab_hbm, utab_hbm, gtab_hbm,
        mout_hbm, uout_hbm, gout_hbm,
        mi_v, ui_v, gi_v, tab_v, mrows, urows, gavg,
        sem_m, sem_u, sem_t, sem_i):
    wid = lax.axis_index("s") * NC + lax.axis_index("c")
    base = wid * BW

    cp_t = pltpu.async_copy(gtab_hbm, tab_v, sem_t)
    cp_i1 = pltpu.async_copy(mi_hbm.at[pl.ds(base, BW)], mi_v, sem_i)
    cp_i2 = pltpu.async_copy(ui_hbm.at[pl.ds(base, BW)], ui_v, sem_i)
    cp_i3 = pltpu.async_copy(mg_hbm.at[pl.ds(base, BW)], gi_v, sem_i)

    cp_i1.wait()
    cp_m = pltpu.async_copy(mtab_hbm.at[mi_v], mrows, sem_m)
    cp_i2.wait()
    cp_u = pltpu.async_copy(utab_hbm.at[ui_v], urows, sem_u)
    cp_i3.wait()
    cp_t.wait()

    one = jnp.ones((16,), jnp.float32)
    fzero = jnp.zeros((16,), jnp.float32)
    lanes = lax.iota(jnp.int32, 16)
    hi = lanes + 16

    def one_row(i):
      iv = jnp.full((16,), i, jnp.int32)
      acc0 = jnp.zeros((16,), jnp.float32)
      acc1 = jnp.zeros((16,), jnp.float32)
      cnt = jnp.zeros((16,), jnp.float32)
      for l in range(L):
        ids = plsc.load_gather(gi_v, [iv, jnp.full((16,), l, jnp.int32)])
        acc0 = acc0 + plsc.load_gather(tab_v, [ids, lanes])
        acc1 = acc1 + plsc.load_gather(tab_v, [ids, hi])
        cnt = cnt + jnp.where(ids != 0, one, fzero)
      r = 1.0 / cnt
      gavg[i, pl.ds(0, 16)] = acc0 * r
      gavg[i, pl.ds(16, 16)] = acc1 * r

    plsc.parallel_loop(0, BW, unroll=2)(one_row)

    cp_m.wait()
    om = pltpu.async_copy(mrows, mout_hbm.at[pl.ds(base, BW)], sem_i)
    cp_u.wait()
    ou = pltpu.async_copy(urows, uout_hbm.at[pl.ds(base, BW)], sem_i)
    og = pltpu.async_copy(
        gavg, gout_hbm.at[pl.ds(base, BW), pl.ds(0, GEMB)], sem_i)
    om.wait()
    ou.wait()
    og.wait()

  return k(mi, ui, mg, mtab, utab, gtab)


def _mlp_body(m_ref, u_ref, g_ref, w1_ref, b1_ref, w2_ref, b2_ref, out_ref):
  h = jnp.dot(m_ref[...], w1_ref[0:EMB, :], preferred_element_type=jnp.float32)
  h = h + jnp.dot(u_ref[...], w1_ref[EMB:2 * EMB, :],
                  preferred_element_type=jnp.float32)
  h = h + jnp.dot(g_ref[:, 0:GEMB], w1_ref[2 * EMB:, :],
                  preferred_element_type=jnp.float32)
  h = jnp.maximum(h + b1_ref[...], 0.0)
  o = jnp.dot(h, w2_ref[...], preferred_element_type=jnp.float32)
  out_ref[...] = jnp.maximum(o + b2_ref[...], 0.0)


def _mlp(movie_e, user_e, genre_avg, W1, b1, W2, b2):
  BB = 2048
  grid = (B // BB,)
  return pl.pallas_call(
      _mlp_body,
      grid=grid,
      in_specs=[
          pl.BlockSpec((BB, EMB), lambda i: (i, 0)),
          pl.BlockSpec((BB, EMB), lambda i: (i, 0)),
          pl.BlockSpec((BB, EMB), lambda i: (i, 0)),
          pl.BlockSpec((EMB + EMB + GEMB, H1), lambda i: (0, 0)),
          pl.BlockSpec((H1,), lambda i: (0,)),
          pl.BlockSpec((H1, H2), lambda i: (0, 0)),
          pl.BlockSpec((H2,), lambda i: (0,)),
      ],
      out_specs=pl.BlockSpec((BB, H2), lambda i: (i, 0)),
      out_shape=jax.ShapeDtypeStruct((B, H2), jnp.float32),
  )(movie_e, user_e, genre_avg, W1, b1, W2, b2)


def kernel(movie_id, user_id, movie_genres, movie_table, user_table,
           genre_table, W1, b1, W2, b2):
  movie_e, user_e, genre_avg = _sc_gather_pool(
      movie_id, user_id, movie_genres, movie_table, user_table, genre_table)

  return _mlp(movie_e, user_e, genre_avg, W1, b1, W2, b2)


# speedup vs baseline: 1.0093x; 1.0047x over previous
import functools

import jax
import jax.numpy as jnp
from jax import lax
from jax.experimental import pallas as pl
from jax.experimental.pallas import tpu as pltpu
from jax.experimental.pallas import tpu_sc as plsc

B = 4096
L = 20
EMB = 128
GEMB = 32
GV = 1000
H1 = 256
H2 = 128

NC = 2
NS = 16
NW = NC * NS
BW = B // NW


def _sc_gather_pool(mi, ui, mg, mtab, utab, gtab):
  mesh = plsc.VectorSubcoreMesh(core_axis_name="c", subcore_axis_name="s")

  @functools.partial(
      pl.kernel,
      mesh=mesh,
      compiler_params=pltpu.CompilerParams(
          use_tc_tiling_on_sc=False, needs_layout_passes=False),
      out_type=[
          jax.ShapeDtypeStruct((B, EMB), jnp.float32),
          jax.ShapeDtypeStruct((B, EMB), jnp.float32),
          jax.ShapeDtypeStruct((B, EMB), jnp.float32),
      ],
      scratch_types=[
          pltpu.VMEM((BW,), jnp.int32),
          pltpu.VMEM((BW,), jnp.int32),
          pltpu.VMEM((BW, L), jnp.int32),
          pltpu.VMEM((GV, GEMB), jnp.float32),
          pltpu.VMEM((BW, EMB), jnp.float32),
          pltpu.VMEM((BW, EMB), jnp.float32),
          pltpu.VMEM((BW, GEMB), jnp.float32),
          pltpu.SemaphoreType.DMA,
          pltpu.SemaphoreType.DMA,
          pltpu.SemaphoreType.DMA,
          pltpu.SemaphoreType.DMA,
      ],
  )
  def k(mi_hbm, ui_hbm, mg_hbm, mtab_hbm, utab_hbm, gtab_hbm,
        mout_hbm, uout_hbm, gout_hbm,
        mi_v, ui_v, gi_v, tab_v, mrows, urows, gavg,
        sem_m, sem_u, sem_t, sem_i):
    wid = lax.axis_index("s") * NC + lax.axis_index("c")
    base = wid * BW

    cp_t = pltpu.async_copy(gtab_hbm, tab_v, sem_t)
    cp_i1 = pltpu.async_copy(mi_hbm.at[pl.ds(base, BW)], mi_v, sem_i)
    cp_i2 = pltpu.async_copy(ui_hbm.at[pl.ds(base, BW)], ui_v, sem_i)
    cp_i3 = pltpu.async_copy(mg_hbm.at[pl.ds(base, BW)], gi_v, sem_i)

    cp_i1.wait()
    cp_m = pltpu.async_copy(mtab_hbm.at[mi_v], mrows, sem_m)
    cp_i2.wait()
    cp_u = pltpu.async_copy(utab_hbm.at[ui_v], urows, sem_u)
    cp_i3.wait()
    cp_t.wait()

    one = jnp.ones((16,), jnp.float32)
    fzero = jnp.zeros((16,), jnp.float32)
    lanes = lax.iota(jnp.int32, 16)
    hi = lanes + 16

    def one_row(i):
      iv = jnp.full((16,), i, jnp.int32)
      acc0 = jnp.zeros((16,), jnp.float32)
      acc1 = jnp.zeros((16,), jnp.float32)
      cnt = jnp.zeros((16,), jnp.float32)
      for l in range(L):
        ids = plsc.load_gather(gi_v, [iv, jnp.full((16,), l, jnp.int32)])
        acc0 = acc0 + plsc.load_gather(tab_v, [ids, lanes])
        acc1 = acc1 + plsc.load_gather(tab_v, [ids, hi])
        cnt = cnt + jnp.where(ids != 0, one, fzero)
      r = 1.0 / cnt
      gavg[i, pl.ds(0, 16)] = acc0 * r
      gavg[i, pl.ds(16, 16)] = acc1 * r

    plsc.parallel_loop(0, BW, unroll=2)(one_row)

    cp_m.wait()
    om = pltpu.async_copy(mrows, mout_hbm.at[pl.ds(base, BW)], sem_i)
    cp_u.wait()
    ou = pltpu.async_copy(urows, uout_hbm.at[pl.ds(base, BW)], sem_i)
    og = pltpu.async_copy(
        gavg, gout_hbm.at[pl.ds(base, BW), pl.ds(0, GEMB)], sem_i)
    om.wait()
    ou.wait()
    og.wait()

  return k(mi, ui, mg, mtab, utab, gtab)


def _mlp_body(m_ref, u_ref, g_ref, w1_ref, b1_ref, w2_ref, b2_ref, out_ref):
  bf = jnp.bfloat16
  w1 = w1_ref[...].astype(bf)
  h = jnp.dot(m_ref[...].astype(bf), w1[0:EMB, :],
              preferred_element_type=jnp.float32)
  h = h + jnp.dot(u_ref[...].astype(bf), w1[EMB:2 * EMB, :],
                  preferred_element_type=jnp.float32)
  h = h + jnp.dot(g_ref[:, 0:GEMB].astype(bf), w1[2 * EMB:, :],
                  preferred_element_type=jnp.float32)
  h = jnp.maximum(h + b1_ref[...], 0.0)
  o = jnp.dot(h.astype(bf), w2_ref[...].astype(bf),
              preferred_element_type=jnp.float32)
  out_ref[...] = jnp.maximum(o + b2_ref[...], 0.0)


def _mlp(movie_e, user_e, genre_avg, W1, b1, W2, b2):
  BB = 2048
  grid = (B // BB,)
  return pl.pallas_call(
      _mlp_body,
      grid=grid,
      in_specs=[
          pl.BlockSpec((BB, EMB), lambda i: (i, 0)),
          pl.BlockSpec((BB, EMB), lambda i: (i, 0)),
          pl.BlockSpec((BB, EMB), lambda i: (i, 0)),
          pl.BlockSpec((EMB + EMB + GEMB, H1), lambda i: (0, 0)),
          pl.BlockSpec((H1,), lambda i: (0,)),
          pl.BlockSpec((H1, H2), lambda i: (0, 0)),
          pl.BlockSpec((H2,), lambda i: (0,)),
      ],
      out_specs=pl.BlockSpec((BB, H2), lambda i: (i, 0)),
      out_shape=jax.ShapeDtypeStruct((B, H2), jnp.float32),
  )(movie_e, user_e, genre_avg, W1, b1, W2, b2)


def kernel(movie_id, user_id, movie_genres, movie_table, user_table,
           genre_table, W1, b1, W2, b2):
  movie_e, user_e, genre_avg = _sc_gather_pool(
      movie_id, user_id, movie_genres, movie_table, user_table, genre_table)

  return _mlp(movie_e, user_e, genre_avg, W1, b1, W2, b2)
